# baseline (speedup 1.0000x reference)
import jax
import jax.numpy as jnp
from jax.experimental import pallas as pl

_BM = 128


def _matmul_kernel(w_ref, t_ref, o_ref):
    o_ref[...] = jnp.dot(w_ref[...], t_ref[...],
                         preferred_element_type=jnp.float32)


def kernel(weights, table):
    B, V = weights.shape
    D = table.shape[1]
    grid = (B // _BM,)
    return pl.pallas_call(
        _matmul_kernel,
        grid=grid,
        in_specs=[
            pl.BlockSpec((_BM, V), lambda i: (i, 0)),
            pl.BlockSpec((V, D), lambda i: (0, 0)),
        ],
        out_specs=pl.BlockSpec((_BM, D), lambda i: (i, 0)),
        out_shape=jax.ShapeDtypeStruct((B, D), jnp.float32),
    )(weights, table)


# speedup vs baseline: 1.3591x; 1.3591x over previous
import jax
import jax.numpy as jnp
from jax.experimental import pallas as pl

_BM = 512


def _matmul_kernel(w_ref, t_ref, o_ref):
    o_ref[...] = jnp.dot(w_ref[...], t_ref[...],
                         preferred_element_type=jnp.float32)


def kernel(weights, table):
    B, V = weights.shape
    D = table.shape[1]
    grid = (B // _BM,)
    return pl.pallas_call(
        _matmul_kernel,
        grid=grid,
        in_specs=[
            pl.BlockSpec((_BM, V), lambda i: (i, 0)),
            pl.BlockSpec((V, D), lambda i: (0, 0)),
        ],
        out_specs=pl.BlockSpec((_BM, D), lambda i: (i, 0)),
        out_shape=jax.ShapeDtypeStruct((B, D), jnp.float32),
    )(weights, table)
